# Initial kernel scaffold; baseline (speedup 1.0000x reference)
import jax
import jax.numpy as jnp
from jax.experimental import pallas as pl


def kernel(node_feats, node_attrs, edge_embedding, edge_attrs, edge_index, batch, W1, W2, W3, W4):
    raise NotImplementedError("write your pallas kernel here")


# speedup vs baseline: 2.0132x; 2.0132x over previous
import functools

import jax
import jax.numpy as jnp
from jax import lax
from jax.experimental import pallas as pl
from jax.experimental.pallas import tpu as pltpu
from jax.experimental.pallas import tpu_sc as plsc

N_NODES = 10000
N_EDGES = 160000
D_FEAT = 128
D_EEMB = 16
D_ATTR = 4

NC = 2
NS = 16
NW = NC * NS

CH = 128
E_PAD = 163840
EPW = E_PAD // NW
NCHUNK = EPW // CH
EPC = E_PAD // NC
EPT = EPC // NS
N_PAD = 10240
RPT = N_PAD // NS

INV1 = 1.0 / float((272 * 4) ** 0.5)
INV2 = 1.0 / float((128 * 4) ** 0.5)
INV3 = 1.0 / float((256 * 4) ** 0.5)
INV4 = 1.0 / float((512) ** 0.5)

def _gather_sc_body(feats, dst2, src2, xi_out, xj_out, idx_d, idx_s, rows_d, rows_s, sem_d, sem_s):
    wid = lax.axis_index("s") * NC + lax.axis_index("c")
    rbase = wid * NCHUNK
    ebase = wid * EPW
    pltpu.sync_copy(dst2.at[pl.ds(rbase, NCHUNK)], idx_d)
    pltpu.sync_copy(src2.at[pl.ds(rbase, NCHUNK)], idx_s)

    @pl.loop(0, NCHUNK)
    def _chunk(j):
        cp_d = pltpu.async_copy(feats.at[idx_d.at[j]], rows_d, sem_d)
        cp_s = pltpu.async_copy(feats.at[idx_s.at[j]], rows_s, sem_s)
        cp_d.wait()
        cp_s.wait()
        pltpu.sync_copy(rows_d, xi_out.at[pl.ds(ebase + j * CH, CH)])
        pltpu.sync_copy(rows_s, xj_out.at[pl.ds(ebase + j * CH, CH)])


def _scatter_sc_body(msgs, dst2, zeros, out, shared, idx_v, rows_v):
    c = lax.axis_index("c")
    s = lax.axis_index("s")
    pltpu.sync_copy(zeros.at[pl.ds(s * RPT, RPT)], shared.at[pl.ds(s * RPT, RPT)])
    rbase = c * (EPC // CH) + s * NCHUNK
    pltpu.sync_copy(dst2.at[pl.ds(rbase, NCHUNK)], idx_v)
    plsc.subcore_barrier()
    ebase = c * EPC + s * EPT

    @pl.loop(0, NCHUNK)
    def _chunk(j):
        pltpu.sync_copy(msgs.at[pl.ds(ebase + j * CH, CH)], rows_v)
        pltpu.sync_copy(rows_v, shared.at[idx_v.at[j]], add=True)

    plsc.subcore_barrier()
    pltpu.sync_copy(shared.at[pl.ds(s * RPT, RPT)], out.at[c, pl.ds(s * RPT, RPT)])


@functools.lru_cache(maxsize=None)
def _sc_kernels():
    mesh = plsc.VectorSubcoreMesh(
        core_axis_name="c", subcore_axis_name="s", num_cores=NC, num_subcores=NS
    )
    gather = pl.kernel(
        _gather_sc_body,
        out_type=(
            jax.ShapeDtypeStruct((E_PAD, D_FEAT), jnp.float32),
            jax.ShapeDtypeStruct((E_PAD, D_FEAT), jnp.float32),
        ),
        mesh=mesh,
        scratch_types=[
            pltpu.VMEM((NCHUNK, CH), jnp.int32),
            pltpu.VMEM((NCHUNK, CH), jnp.int32),
            pltpu.VMEM((CH, D_FEAT), jnp.float32),
            pltpu.VMEM((CH, D_FEAT), jnp.float32),
            pltpu.SemaphoreType.DMA,
            pltpu.SemaphoreType.DMA,
        ],
    )
    scatter = pl.kernel(
        _scatter_sc_body,
        out_type=jax.ShapeDtypeStruct((NC, N_PAD, D_FEAT), jnp.float32),
        mesh=mesh,
        scratch_types=[
            pltpu.VMEM_SHARED((N_PAD, D_FEAT), jnp.float32),
            pltpu.VMEM((NCHUNK, CH), jnp.int32),
            pltpu.VMEM((CH, D_FEAT), jnp.float32),
        ],
    )
    return gather, scatter


BLK_E = 1024


def _silu(x):
    return x * jax.nn.sigmoid(x)


def _msg_body(xi_ref, xj_ref, emb_ref, att_ref, w1a, w1b, w1c, w2, out_ref):
    xi = xi_ref[...]
    xj = xj_ref[...]
    emb = emb_ref[...]
    a = att_ref[...]
    acc = jnp.zeros((BLK_E, D_FEAT), jnp.float32)
    for g in range(D_ATTR):
        t = (
            jnp.dot(xi, w1a[g], preferred_element_type=jnp.float32)
            + jnp.dot(xj, w1b[g], preferred_element_type=jnp.float32)
            + jnp.dot(emb, w1c[g], preferred_element_type=jnp.float32)
        )
        acc = acc + a[:, g : g + 1] * t
    m1 = _silu(acc * INV1)
    acc2 = jnp.zeros((BLK_E, D_FEAT), jnp.float32)
    for g in range(D_ATTR):
        acc2 = acc2 + a[:, g : g + 1] * jnp.dot(
            m1, w2[g], preferred_element_type=jnp.float32
        )
    out_ref[...] = _silu(acc2 * INV2)


def _messages_tc(xi, xj, emb, att, w1a, w1b, w1c, w2):
    n_blk = E_PAD // BLK_E
    return pl.pallas_call(
        _msg_body,
        grid=(n_blk,),
        in_specs=[
            pl.BlockSpec((BLK_E, D_FEAT), lambda i: (i, 0)),
            pl.BlockSpec((BLK_E, D_FEAT), lambda i: (i, 0)),
            pl.BlockSpec((BLK_E, D_EEMB), lambda i: (i, 0)),
            pl.BlockSpec((BLK_E, D_ATTR), lambda i: (i, 0)),
            pl.BlockSpec((D_ATTR, D_FEAT, D_FEAT), lambda i: (0, 0, 0)),
            pl.BlockSpec((D_ATTR, D_FEAT, D_FEAT), lambda i: (0, 0, 0)),
            pl.BlockSpec((D_ATTR, D_EEMB, D_FEAT), lambda i: (0, 0, 0)),
            pl.BlockSpec((D_ATTR, D_FEAT, D_FEAT), lambda i: (0, 0, 0)),
        ],
        out_specs=pl.BlockSpec((BLK_E, D_FEAT), lambda i: (i, 0)),
        out_shape=jax.ShapeDtypeStruct((E_PAD, D_FEAT), jnp.float32),
    )(xi, xj, emb, att, w1a, w1b, w1c, w2)


BLK_N = 2000


def _upd_body(f_ref, pa_ref, pb_ref, na_ref, w3a, w3b, w4, out_ref):
    f = f_ref[...]
    msg = pa_ref[...] + pb_ref[...]
    na = na_ref[...]
    acc = jnp.zeros((BLK_N, D_FEAT), jnp.float32)
    for g in range(D_ATTR):
        t = jnp.dot(f, w3a[g], preferred_element_type=jnp.float32) + jnp.dot(
            msg, w3b[g], preferred_element_type=jnp.float32
        )
        acc = acc + na[:, g : g + 1] * t
    u = _silu(acc * INV3)
    acc2 = jnp.zeros((BLK_N, D_FEAT), jnp.float32)
    for g in range(D_ATTR):
        acc2 = acc2 + na[:, g : g + 1] * jnp.dot(
            u, w4[g], preferred_element_type=jnp.float32
        )
    out_ref[...] = acc2 * INV4 + f


def _update_tc(feats, pa, pb, nattr, w3a, w3b, w4):
    n_blk = N_NODES // BLK_N
    return pl.pallas_call(
        _upd_body,
        grid=(n_blk,),
        in_specs=[
            pl.BlockSpec((BLK_N, D_FEAT), lambda i: (i, 0)),
            pl.BlockSpec((BLK_N, D_FEAT), lambda i: (i, 0)),
            pl.BlockSpec((BLK_N, D_FEAT), lambda i: (i, 0)),
            pl.BlockSpec((BLK_N, D_ATTR), lambda i: (i, 0)),
            pl.BlockSpec((D_ATTR, D_FEAT, D_FEAT), lambda i: (0, 0, 0)),
            pl.BlockSpec((D_ATTR, D_FEAT, D_FEAT), lambda i: (0, 0, 0)),
            pl.BlockSpec((D_ATTR, D_FEAT, D_FEAT), lambda i: (0, 0, 0)),
        ],
        out_specs=pl.BlockSpec((BLK_N, D_FEAT), lambda i: (i, 0)),
        out_shape=jax.ShapeDtypeStruct((N_NODES, D_FEAT), jnp.float32),
    )(feats, pa, pb, nattr, w3a, w3b, w4)


def kernel(node_feats, node_attrs, edge_embedding, edge_attrs, edge_index, batch, W1, W2, W3, W4):
    del batch
    pad = E_PAD - N_EDGES
    src = edge_index[0].astype(jnp.int32)
    dst = edge_index[1].astype(jnp.int32)
    zpad_i = jnp.zeros((pad,), jnp.int32)
    dst2 = jnp.concatenate([dst, zpad_i]).reshape(E_PAD // CH, CH)
    src2 = jnp.concatenate([src, zpad_i]).reshape(E_PAD // CH, CH)
    att_p = jnp.concatenate([edge_attrs, jnp.zeros((pad, D_ATTR), jnp.float32)])
    emb_p = jnp.concatenate([edge_embedding, jnp.zeros((pad, D_EEMB), jnp.float32)])

    w1a = jnp.transpose(W1[:D_FEAT], (1, 0, 2))
    w1b = jnp.transpose(W1[D_FEAT : 2 * D_FEAT], (1, 0, 2))
    w1c = jnp.transpose(W1[2 * D_FEAT :], (1, 0, 2))
    w2 = jnp.transpose(W2, (1, 0, 2))
    w3a = jnp.transpose(W3[:D_FEAT], (1, 0, 2))
    w3b = jnp.transpose(W3[D_FEAT:], (1, 0, 2))
    w4 = jnp.transpose(W4, (1, 0, 2))

    gather_sc, scatter_sc = _sc_kernels()
    xi, xj = gather_sc(node_feats, dst2, src2)
    msgs = _messages_tc(xi, xj, emb_p, att_p, w1a, w1b, w1c, w2)
    partials = scatter_sc(msgs, dst2, jnp.zeros((N_PAD, D_FEAT), jnp.float32))
    return _update_tc(node_feats, partials[0, :N_NODES], partials[1, :N_NODES], node_attrs, w3a, w3b, w4)


# speedup vs baseline: 2.0989x; 1.0426x over previous
import functools

import jax
import jax.numpy as jnp
from jax import lax
from jax.experimental import pallas as pl
from jax.experimental.pallas import tpu as pltpu
from jax.experimental.pallas import tpu_sc as plsc

N_NODES = 10000
N_EDGES = 160000
D_FEAT = 128
D_EEMB = 16
D_ATTR = 4

NC = 2
NS = 16
NW = NC * NS

CH = 128
E_PAD = 163840
EPW = E_PAD // NW
NCHUNK = EPW // CH
EPC = E_PAD // NC
EPT = EPC // NS
N_PAD = 10240
RPT = N_PAD // NS

INV1 = 1.0 / float((272 * 4) ** 0.5)
INV2 = 1.0 / float((128 * 4) ** 0.5)
INV3 = 1.0 / float((256 * 4) ** 0.5)
INV4 = 1.0 / float((512) ** 0.5)

NBUF = 4


def _gather_sc_body(feats, dst2, src2, xi_out, xj_out, idx_d, idx_s, rows, sem_g, sem_w):
    wid = lax.axis_index("s") * NC + lax.axis_index("c")
    rbase = wid * NCHUNK
    ebase = wid * EPW
    pltpu.sync_copy(dst2.at[pl.ds(rbase, NCHUNK)], idx_d)
    pltpu.sync_copy(src2.at[pl.ds(rbase, NCHUNK)], idx_s)

    @pl.loop(0, NCHUNK, step=2)
    def _sup(j0):
        gathers = []
        for b in range(NBUF):
            j = j0 + b // 2
            idx = idx_d if b % 2 == 0 else idx_s
            gathers.append(
                pltpu.async_copy(feats.at[idx.at[j]], rows.at[b], sem_g)
            )
        @pl.when(j0 > 0)
        def _():
            for b in range(NBUF):
                out = xi_out if b % 2 == 0 else xj_out
                pltpu.make_async_copy(
                    out.at[pl.ds(0, CH)], rows.at[b], sem_w
                ).wait()

        for b in range(NBUF):
            j = j0 + b // 2
            out = xi_out if b % 2 == 0 else xj_out
            gathers[b].wait()
            pltpu.async_copy(rows.at[b], out.at[pl.ds(ebase + j * CH, CH)], sem_w)

    for b in range(NBUF):
        out = xi_out if b % 2 == 0 else xj_out
        pltpu.make_async_copy(out.at[pl.ds(0, CH)], rows.at[b], sem_w).wait()


def _scatter_sc_body(msgs, dst2, zeros, out, shared, idx_v, rows_v):
    c = lax.axis_index("c")
    s = lax.axis_index("s")
    pltpu.sync_copy(zeros.at[pl.ds(s * RPT, RPT)], shared.at[pl.ds(s * RPT, RPT)])
    rbase = c * (EPC // CH) + s * NCHUNK
    pltpu.sync_copy(dst2.at[pl.ds(rbase, NCHUNK)], idx_v)
    plsc.subcore_barrier()
    ebase = c * EPC + s * EPT

    @pl.loop(0, NCHUNK)
    def _chunk(j):
        pltpu.sync_copy(msgs.at[pl.ds(ebase + j * CH, CH)], rows_v)
        pltpu.sync_copy(rows_v, shared.at[idx_v.at[j]], add=True)

    plsc.subcore_barrier()
    pltpu.sync_copy(shared.at[pl.ds(s * RPT, RPT)], out.at[c, pl.ds(s * RPT, RPT)])


@functools.lru_cache(maxsize=None)
def _sc_kernels():
    mesh = plsc.VectorSubcoreMesh(
        core_axis_name="c", subcore_axis_name="s", num_cores=NC, num_subcores=NS
    )
    gather = pl.kernel(
        _gather_sc_body,
        out_type=(
            jax.ShapeDtypeStruct((E_PAD, D_FEAT), jnp.float32),
            jax.ShapeDtypeStruct((E_PAD, D_FEAT), jnp.float32),
        ),
        mesh=mesh,
        scratch_types=[
            pltpu.VMEM((NCHUNK, CH), jnp.int32),
            pltpu.VMEM((NCHUNK, CH), jnp.int32),
            pltpu.VMEM((NBUF, CH, D_FEAT), jnp.float32),
            pltpu.SemaphoreType.DMA,
            pltpu.SemaphoreType.DMA,
        ],
    )
    scatter = pl.kernel(
        _scatter_sc_body,
        out_type=jax.ShapeDtypeStruct((NC, N_PAD, D_FEAT), jnp.float32),
        mesh=mesh,
        scratch_types=[
            pltpu.VMEM_SHARED((N_PAD, D_FEAT), jnp.float32),
            pltpu.VMEM((NCHUNK, CH), jnp.int32),
            pltpu.VMEM((CH, D_FEAT), jnp.float32),
        ],
    )
    return gather, scatter


BLK_E = 1024


def _silu(x):
    return x * jax.nn.sigmoid(x)


def _msg_body(xi_ref, xj_ref, emb_ref, att_ref, w1a, w1b, w1c, w2, out_ref):
    xi = xi_ref[...]
    xj = xj_ref[...]
    emb = emb_ref[...]
    a = att_ref[...]
    acc = jnp.zeros((BLK_E, D_FEAT), jnp.float32)
    for g in range(D_ATTR):
        t = (
            jnp.dot(xi, w1a[g], preferred_element_type=jnp.float32)
            + jnp.dot(xj, w1b[g], preferred_element_type=jnp.float32)
            + jnp.dot(emb, w1c[g], preferred_element_type=jnp.float32)
        )
        acc = acc + a[:, g : g + 1] * t
    m1 = _silu(acc * INV1)
    acc2 = jnp.zeros((BLK_E, D_FEAT), jnp.float32)
    for g in range(D_ATTR):
        acc2 = acc2 + a[:, g : g + 1] * jnp.dot(
            m1, w2[g], preferred_element_type=jnp.float32
        )
    out_ref[...] = _silu(acc2 * INV2)


def _messages_tc(xi, xj, emb, att, w1a, w1b, w1c, w2):
    n_blk = E_PAD // BLK_E
    return pl.pallas_call(
        _msg_body,
        grid=(n_blk,),
        in_specs=[
            pl.BlockSpec((BLK_E, D_FEAT), lambda i: (i, 0)),
            pl.BlockSpec((BLK_E, D_FEAT), lambda i: (i, 0)),
            pl.BlockSpec((BLK_E, D_EEMB), lambda i: (i, 0)),
            pl.BlockSpec((BLK_E, D_ATTR), lambda i: (i, 0)),
            pl.BlockSpec((D_ATTR, D_FEAT, D_FEAT), lambda i: (0, 0, 0)),
            pl.BlockSpec((D_ATTR, D_FEAT, D_FEAT), lambda i: (0, 0, 0)),
            pl.BlockSpec((D_ATTR, D_EEMB, D_FEAT), lambda i: (0, 0, 0)),
            pl.BlockSpec((D_ATTR, D_FEAT, D_FEAT), lambda i: (0, 0, 0)),
        ],
        out_specs=pl.BlockSpec((BLK_E, D_FEAT), lambda i: (i, 0)),
        out_shape=jax.ShapeDtypeStruct((E_PAD, D_FEAT), jnp.float32),
    )(xi, xj, emb, att, w1a, w1b, w1c, w2)


BLK_N = 2000


def _upd_body(f_ref, pa_ref, pb_ref, na_ref, w3a, w3b, w4, out_ref):
    f = f_ref[...]
    msg = pa_ref[...] + pb_ref[...]
    na = na_ref[...]
    acc = jnp.zeros((BLK_N, D_FEAT), jnp.float32)
    for g in range(D_ATTR):
        t = jnp.dot(f, w3a[g], preferred_element_type=jnp.float32) + jnp.dot(
            msg, w3b[g], preferred_element_type=jnp.float32
        )
        acc = acc + na[:, g : g + 1] * t
    u = _silu(acc * INV3)
    acc2 = jnp.zeros((BLK_N, D_FEAT), jnp.float32)
    for g in range(D_ATTR):
        acc2 = acc2 + na[:, g : g + 1] * jnp.dot(
            u, w4[g], preferred_element_type=jnp.float32
        )
    out_ref[...] = acc2 * INV4 + f


def _update_tc(feats, pa, pb, nattr, w3a, w3b, w4):
    n_blk = N_NODES // BLK_N
    return pl.pallas_call(
        _upd_body,
        grid=(n_blk,),
        in_specs=[
            pl.BlockSpec((BLK_N, D_FEAT), lambda i: (i, 0)),
            pl.BlockSpec((BLK_N, D_FEAT), lambda i: (i, 0)),
            pl.BlockSpec((BLK_N, D_FEAT), lambda i: (i, 0)),
            pl.BlockSpec((BLK_N, D_ATTR), lambda i: (i, 0)),
            pl.BlockSpec((D_ATTR, D_FEAT, D_FEAT), lambda i: (0, 0, 0)),
            pl.BlockSpec((D_ATTR, D_FEAT, D_FEAT), lambda i: (0, 0, 0)),
            pl.BlockSpec((D_ATTR, D_FEAT, D_FEAT), lambda i: (0, 0, 0)),
        ],
        out_specs=pl.BlockSpec((BLK_N, D_FEAT), lambda i: (i, 0)),
        out_shape=jax.ShapeDtypeStruct((N_NODES, D_FEAT), jnp.float32),
    )(feats, pa, pb, nattr, w3a, w3b, w4)


def kernel(node_feats, node_attrs, edge_embedding, edge_attrs, edge_index, batch, W1, W2, W3, W4):
    del batch
    pad = E_PAD - N_EDGES
    src = edge_index[0].astype(jnp.int32)
    dst = edge_index[1].astype(jnp.int32)
    zpad_i = jnp.zeros((pad,), jnp.int32)
    dst2 = jnp.concatenate([dst, zpad_i]).reshape(E_PAD // CH, CH)
    src2 = jnp.concatenate([src, zpad_i]).reshape(E_PAD // CH, CH)
    att_p = jnp.concatenate([edge_attrs, jnp.zeros((pad, D_ATTR), jnp.float32)])
    emb_p = jnp.concatenate([edge_embedding, jnp.zeros((pad, D_EEMB), jnp.float32)])

    w1a = jnp.transpose(W1[:D_FEAT], (1, 0, 2))
    w1b = jnp.transpose(W1[D_FEAT : 2 * D_FEAT], (1, 0, 2))
    w1c = jnp.transpose(W1[2 * D_FEAT :], (1, 0, 2))
    w2 = jnp.transpose(W2, (1, 0, 2))
    w3a = jnp.transpose(W3[:D_FEAT], (1, 0, 2))
    w3b = jnp.transpose(W3[D_FEAT:], (1, 0, 2))
    w4 = jnp.transpose(W4, (1, 0, 2))

    gather_sc, scatter_sc = _sc_kernels()
    xi, xj = gather_sc(node_feats, dst2, src2)
    msgs = _messages_tc(xi, xj, emb_p, att_p, w1a, w1b, w1c, w2)
    partials = scatter_sc(msgs, dst2, jnp.zeros((N_PAD, D_FEAT), jnp.float32))
    return _update_tc(node_feats, partials[0, :N_NODES], partials[1, :N_NODES], node_attrs, w3a, w3b, w4)


# speedup vs baseline: 2.5669x; 1.2229x over previous
import functools

import jax
import jax.numpy as jnp
from jax import lax
from jax.experimental import pallas as pl
from jax.experimental.pallas import tpu as pltpu
from jax.experimental.pallas import tpu_sc as plsc

N_NODES = 10000
N_EDGES = 160000
D_FEAT = 128
D_EEMB = 16
D_ATTR = 4

NC = 2
NS = 16
NW = NC * NS

CH = 128
E_PAD = 163840
NCH_PAD = E_PAD // CH
NCHUNK = 40
N_PAD = 10240
RPT = N_PAD // NS
DP = D_FEAT // 2

INV1 = 1.0 / float((272 * 4) ** 0.5)
INV2 = 1.0 / float((128 * 4) ** 0.5)
INV3 = 1.0 / float((256 * 4) ** 0.5)
INV4 = 1.0 / float((512) ** 0.5)


NBUF = 4


def _gather_sc_body(feats, dst2, src2, xi_out, xj_out, idx_d, idx_s, rows, sem_g, sem_w):
    wid = lax.axis_index("s") * NC + lax.axis_index("c")
    cs = wid * NCHUNK
    pltpu.sync_copy(dst2.at[pl.ds(cs, NCHUNK)], idx_d)
    pltpu.sync_copy(src2.at[pl.ds(cs, NCHUNK)], idx_s)

    def _fire(j, b):
        idx = idx_d if b % 2 == 0 else idx_s
        return pltpu.async_copy(feats.at[idx.at[j]], rows.at[b], sem_g)

    def _write(j, b):
        out = xi_out if b % 2 == 0 else xj_out
        return pltpu.async_copy(
            rows.at[b], out.at[pl.ds((cs + j) * CH, CH)], sem_w
        )

    def _drain_write(b):
        pltpu.make_async_copy(xi_out.at[pl.ds(0, CH)], rows.at[b], sem_w).wait()

    @pl.loop(0, NCHUNK, step=2)
    def _sup(j0):
        gathers = [_fire(j0 + b // 2, b) for b in range(NBUF)]

        @pl.when(j0 > 0)
        def _():
            for b in range(NBUF):
                _drain_write(b)

        for b in range(NBUF):
            gathers[b].wait()
            _write(j0 + b // 2, b)

    for b in range(NBUF):
        _drain_write(b)


def _scatter_sc_body(msgs, dst2, zeros, out, shared, idx_v, rows_v):
    c = lax.axis_index("c")
    s = lax.axis_index("s")
    pltpu.sync_copy(zeros.at[pl.ds(s * RPT, RPT)], shared.at[pl.ds(s * RPT, RPT)])
    cs = (c * NS + s) * NCHUNK
    pltpu.sync_copy(dst2.at[pl.ds(cs, NCHUNK)], idx_v)
    plsc.subcore_barrier()

    @pl.loop(0, NCHUNK)
    def _chunk(j):
        pltpu.sync_copy(msgs.at[pl.ds((cs + j) * CH, CH)], rows_v)
        pltpu.sync_copy(rows_v, shared.at[idx_v.at[j]], add=True)

    plsc.subcore_barrier()
    pltpu.sync_copy(shared.at[pl.ds(s * RPT, RPT)], out.at[c, pl.ds(s * RPT, RPT)])


@functools.lru_cache(maxsize=None)
def _sc_kernels():
    mesh = plsc.VectorSubcoreMesh(
        core_axis_name="c", subcore_axis_name="s", num_cores=NC, num_subcores=NS
    )
    gather = pl.kernel(
        _gather_sc_body,
        compiler_params=pltpu.CompilerParams(use_tc_tiling_on_sc=False),
        out_type=(
            jax.ShapeDtypeStruct((E_PAD, DP), jnp.int32),
            jax.ShapeDtypeStruct((E_PAD, DP), jnp.int32),
        ),
        mesh=mesh,
        scratch_types=[
            pltpu.VMEM((NCHUNK, CH), jnp.int32),
            pltpu.VMEM((NCHUNK, CH), jnp.int32),
            pltpu.VMEM((NBUF, CH, DP), jnp.int32),
            pltpu.SemaphoreType.DMA,
            pltpu.SemaphoreType.DMA,
        ],
    )
    scatter = pl.kernel(
        _scatter_sc_body,
        out_type=jax.ShapeDtypeStruct((NC, N_PAD, D_FEAT), jnp.float32),
        mesh=mesh,
        scratch_types=[
            pltpu.VMEM_SHARED((N_PAD, D_FEAT), jnp.float32),
            pltpu.VMEM((NCHUNK, CH), jnp.int32),
            pltpu.VMEM((CH, D_FEAT), jnp.float32),
        ],
    )
    return gather, scatter


BLK_E = 1024


def _silu(x):
    return x * jax.nn.sigmoid(x)


def _unpack_bf16(x):
    lo = lax.bitcast_convert_type(x << 16, jnp.float32)
    hi = lax.bitcast_convert_type(x & jnp.int32(-65536), jnp.float32)
    return jnp.concatenate(
        [lo.astype(jnp.bfloat16), hi.astype(jnp.bfloat16)], axis=1
    )


def _msg_body(xi_ref, xj_ref, emb_ref, att_ref, w1a, w1b, w1c, w2, out_ref):
    xi = _unpack_bf16(xi_ref[...])
    xj = _unpack_bf16(xj_ref[...])
    emb = emb_ref[...]
    a = att_ref[...]
    acc = jnp.zeros((BLK_E, D_FEAT), jnp.float32)
    for g in range(D_ATTR):
        t = (
            jnp.dot(xi, w1a[g], preferred_element_type=jnp.float32)
            + jnp.dot(xj, w1b[g], preferred_element_type=jnp.float32)
            + jnp.dot(emb, w1c[g], preferred_element_type=jnp.float32)
        )
        acc = acc + a[:, g : g + 1] * t
    m1 = _silu(acc * INV1).astype(jnp.bfloat16)
    acc2 = jnp.zeros((BLK_E, D_FEAT), jnp.float32)
    for g in range(D_ATTR):
        acc2 = acc2 + a[:, g : g + 1] * jnp.dot(
            m1, w2[g], preferred_element_type=jnp.float32
        )
    out_ref[...] = _silu(acc2 * INV2)


def _messages_tc(xi, xj, emb, att, w1a, w1b, w1c, w2):
    n_blk = (N_EDGES + BLK_E - 1) // BLK_E
    return pl.pallas_call(
        _msg_body,
        grid=(n_blk,),
        in_specs=[
            pl.BlockSpec((BLK_E, DP), lambda i: (i, 0)),
            pl.BlockSpec((BLK_E, DP), lambda i: (i, 0)),
            pl.BlockSpec((BLK_E, D_EEMB), lambda i: (i, 0)),
            pl.BlockSpec((BLK_E, D_ATTR), lambda i: (i, 0)),
            pl.BlockSpec((D_ATTR, D_FEAT, D_FEAT), lambda i: (0, 0, 0)),
            pl.BlockSpec((D_ATTR, D_FEAT, D_FEAT), lambda i: (0, 0, 0)),
            pl.BlockSpec((D_ATTR, D_EEMB, D_FEAT), lambda i: (0, 0, 0)),
            pl.BlockSpec((D_ATTR, D_FEAT, D_FEAT), lambda i: (0, 0, 0)),
        ],
        out_specs=pl.BlockSpec((BLK_E, D_FEAT), lambda i: (i, 0)),
        out_shape=jax.ShapeDtypeStruct((E_PAD, D_FEAT), jnp.float32),
    )(xi, xj, emb, att, w1a, w1b, w1c, w2)


BLK_N = 2000


def _upd_body(f_ref, pa_ref, pb_ref, na_ref, w3a, w3b, w4, out_ref):
    f = f_ref[...]
    msg = pa_ref[...] + pb_ref[...]
    na = na_ref[...]
    acc = jnp.zeros((BLK_N, D_FEAT), jnp.float32)
    for g in range(D_ATTR):
        t = jnp.dot(f, w3a[g], preferred_element_type=jnp.float32) + jnp.dot(
            msg, w3b[g], preferred_element_type=jnp.float32
        )
        acc = acc + na[:, g : g + 1] * t
    u = _silu(acc * INV3)
    acc2 = jnp.zeros((BLK_N, D_FEAT), jnp.float32)
    for g in range(D_ATTR):
        acc2 = acc2 + na[:, g : g + 1] * jnp.dot(
            u, w4[g], preferred_element_type=jnp.float32
        )
    out_ref[...] = acc2 * INV4 + f


def _update_tc(feats, pa, pb, nattr, w3a, w3b, w4):
    n_blk = N_NODES // BLK_N
    return pl.pallas_call(
        _upd_body,
        grid=(n_blk,),
        in_specs=[
            pl.BlockSpec((BLK_N, D_FEAT), lambda i: (i, 0)),
            pl.BlockSpec((BLK_N, D_FEAT), lambda i: (i, 0)),
            pl.BlockSpec((BLK_N, D_FEAT), lambda i: (i, 0)),
            pl.BlockSpec((BLK_N, D_ATTR), lambda i: (i, 0)),
            pl.BlockSpec((D_ATTR, D_FEAT, D_FEAT), lambda i: (0, 0, 0)),
            pl.BlockSpec((D_ATTR, D_FEAT, D_FEAT), lambda i: (0, 0, 0)),
            pl.BlockSpec((D_ATTR, D_FEAT, D_FEAT), lambda i: (0, 0, 0)),
        ],
        out_specs=pl.BlockSpec((BLK_N, D_FEAT), lambda i: (i, 0)),
        out_shape=jax.ShapeDtypeStruct((N_NODES, D_FEAT), jnp.float32),
    )(feats, pa, pb, nattr, w3a, w3b, w4)


def kernel(node_feats, node_attrs, edge_embedding, edge_attrs, edge_index, batch, W1, W2, W3, W4):
    del batch
    pad = E_PAD - N_EDGES
    src = edge_index[0].astype(jnp.int32)
    dst = edge_index[1].astype(jnp.int32)
    trash = N_PAD - 1
    dst2 = jnp.pad(dst, (0, pad), constant_values=trash).reshape(NCH_PAD, CH)
    src2 = jnp.pad(src, (0, pad), constant_values=trash).reshape(NCH_PAD, CH)

    feats_bf = jnp.pad(
        node_feats.astype(jnp.bfloat16), ((0, N_PAD - N_NODES), (0, 0))
    )
    feats_pk = lax.bitcast_convert_type(feats_bf.reshape(N_PAD, DP, 2), jnp.int32)
    emb_bf = edge_embedding.astype(jnp.bfloat16)
    w1a = jnp.transpose(W1[:D_FEAT], (1, 0, 2)).astype(jnp.bfloat16)
    w1a = jnp.concatenate([w1a[:, 0::2, :], w1a[:, 1::2, :]], axis=1)
    w1b = jnp.transpose(W1[D_FEAT : 2 * D_FEAT], (1, 0, 2)).astype(jnp.bfloat16)
    w1b = jnp.concatenate([w1b[:, 0::2, :], w1b[:, 1::2, :]], axis=1)
    w1c = jnp.transpose(W1[2 * D_FEAT :], (1, 0, 2)).astype(jnp.bfloat16)
    w2 = jnp.transpose(W2, (1, 0, 2)).astype(jnp.bfloat16)
    w3a = jnp.transpose(W3[:D_FEAT], (1, 0, 2))
    w3b = jnp.transpose(W3[D_FEAT:], (1, 0, 2))
    w4 = jnp.transpose(W4, (1, 0, 2))

    gather_sc, scatter_sc = _sc_kernels()
    xi, xj = gather_sc(feats_pk, dst2, src2)
    msgs = _messages_tc(xi, xj, emb_bf, edge_attrs, w1a, w1b, w1c, w2)
    partials = scatter_sc(msgs, dst2, jnp.zeros((N_PAD, D_FEAT), jnp.float32))
    return _update_tc(
        node_feats, partials[0, :N_NODES], partials[1, :N_NODES], node_attrs, w3a, w3b, w4
    )


# speedup vs baseline: 2.6076x; 1.0159x over previous
import functools

import jax
import jax.numpy as jnp
from jax import lax
from jax.experimental import pallas as pl
from jax.experimental.pallas import tpu as pltpu
from jax.experimental.pallas import tpu_sc as plsc

N_NODES = 10000
N_EDGES = 160000
D_FEAT = 128
D_EEMB = 16
D_ATTR = 4

NC = 2
NS = 16
NW = NC * NS

CH = 128
E_PAD = 163840
NCH_PAD = E_PAD // CH
NCHUNK = 40
N_PAD = 10240
RPT = N_PAD // NS
DP = D_FEAT // 2

INV1 = 1.0 / float((272 * 4) ** 0.5)
INV2 = 1.0 / float((128 * 4) ** 0.5)
INV3 = 1.0 / float((256 * 4) ** 0.5)
INV4 = 1.0 / float((512) ** 0.5)


NBUF = 4


def _gather_sc_body(feats, dst2, src2, xi_out, xj_out, idx_d, idx_s, rows, sem_g, sem_w):
    wid = lax.axis_index("s") * NC + lax.axis_index("c")
    cs = wid * NCHUNK
    pltpu.sync_copy(dst2.at[pl.ds(cs, NCHUNK)], idx_d)
    pltpu.sync_copy(src2.at[pl.ds(cs, NCHUNK)], idx_s)

    def _fire(j, b):
        idx = idx_d if b % 2 == 0 else idx_s
        return pltpu.async_copy(feats.at[idx.at[j]], rows.at[b], sem_g)

    def _write(j, b):
        out = xi_out if b % 2 == 0 else xj_out
        return pltpu.async_copy(
            rows.at[b], out.at[pl.ds((cs + j) * CH, CH)], sem_w
        )

    def _drain_write(b):
        pltpu.make_async_copy(xi_out.at[pl.ds(0, CH)], rows.at[b], sem_w).wait()

    @pl.loop(0, NCHUNK, step=2)
    def _sup(j0):
        gathers = [_fire(j0 + b // 2, b) for b in range(NBUF)]

        @pl.when(j0 > 0)
        def _():
            for b in range(NBUF):
                _drain_write(b)

        for b in range(NBUF):
            gathers[b].wait()
            _write(j0 + b // 2, b)

    for b in range(NBUF):
        _drain_write(b)


def _scatter_sc_body(msgs, dst2, zeros, out, shared, idx_v, rows_v):
    c = lax.axis_index("c")
    s = lax.axis_index("s")
    pltpu.sync_copy(zeros.at[pl.ds(s * RPT, RPT)], shared.at[pl.ds(s * RPT, RPT)])
    cs = (c * NS + s) * NCHUNK
    pltpu.sync_copy(dst2.at[pl.ds(cs, NCHUNK)], idx_v)
    plsc.subcore_barrier()

    @pl.loop(0, NCHUNK)
    def _chunk(j):
        pltpu.sync_copy(msgs.at[pl.ds((cs + j) * CH, CH)], rows_v)
        pltpu.sync_copy(rows_v, shared.at[idx_v.at[j]], add=True)

    plsc.subcore_barrier()
    pltpu.sync_copy(shared.at[pl.ds(s * RPT, RPT)], out.at[c, pl.ds(s * RPT, RPT)])


@functools.lru_cache(maxsize=None)
def _sc_kernels():
    mesh = plsc.VectorSubcoreMesh(
        core_axis_name="c", subcore_axis_name="s", num_cores=NC, num_subcores=NS
    )
    gather = pl.kernel(
        _gather_sc_body,
        compiler_params=pltpu.CompilerParams(use_tc_tiling_on_sc=False),
        out_type=(
            jax.ShapeDtypeStruct((E_PAD, DP), jnp.int32),
            jax.ShapeDtypeStruct((E_PAD, DP), jnp.int32),
        ),
        mesh=mesh,
        scratch_types=[
            pltpu.VMEM((NCHUNK, CH), jnp.int32),
            pltpu.VMEM((NCHUNK, CH), jnp.int32),
            pltpu.VMEM((NBUF, CH, DP), jnp.int32),
            pltpu.SemaphoreType.DMA,
            pltpu.SemaphoreType.DMA,
        ],
    )
    scatter = pl.kernel(
        _scatter_sc_body,
        out_type=jax.ShapeDtypeStruct((NC, N_PAD, D_FEAT), jnp.float32),
        mesh=mesh,
        scratch_types=[
            pltpu.VMEM_SHARED((N_PAD, D_FEAT), jnp.float32),
            pltpu.VMEM((NCHUNK, CH), jnp.int32),
            pltpu.VMEM((CH, D_FEAT), jnp.float32),
        ],
    )
    return gather, scatter


BLK_E = 1024


def _silu(x):
    return x * jax.nn.sigmoid(x)


def _unpack_bf16(x):
    lo = lax.bitcast_convert_type(x << 16, jnp.float32)
    hi = lax.bitcast_convert_type(x & jnp.int32(-65536), jnp.float32)
    return jnp.concatenate(
        [lo.astype(jnp.bfloat16), hi.astype(jnp.bfloat16)], axis=1
    )


def _msg_body(xi_ref, xj_ref, emb_ref, att_ref, wab, w1c, w2c, out_ref):
    xi = _unpack_bf16(xi_ref[...])
    xj = _unpack_bf16(xj_ref[...])
    emb = emb_ref[...]
    a = att_ref[...]
    xcat = jnp.concatenate([xi, xj], axis=1)
    t = jnp.dot(xcat, wab[...], preferred_element_type=jnp.float32) + jnp.dot(
        emb, w1c[...], preferred_element_type=jnp.float32
    )
    acc = jnp.zeros((BLK_E, D_FEAT), jnp.float32)
    for g in range(D_ATTR):
        acc = acc + a[:, g : g + 1] * t[:, g * D_FEAT : (g + 1) * D_FEAT]
    m1 = _silu(acc * INV1).astype(jnp.bfloat16)
    t2 = jnp.dot(m1, w2c[...], preferred_element_type=jnp.float32)
    acc2 = jnp.zeros((BLK_E, D_FEAT), jnp.float32)
    for g in range(D_ATTR):
        acc2 = acc2 + a[:, g : g + 1] * t2[:, g * D_FEAT : (g + 1) * D_FEAT]
    out_ref[...] = _silu(acc2 * INV2)


def _messages_tc(xi, xj, emb, att, wab, w1c, w2c):
    n_blk = (N_EDGES + BLK_E - 1) // BLK_E
    return pl.pallas_call(
        _msg_body,
        grid=(n_blk,),
        in_specs=[
            pl.BlockSpec((BLK_E, DP), lambda i: (i, 0)),
            pl.BlockSpec((BLK_E, DP), lambda i: (i, 0)),
            pl.BlockSpec((BLK_E, D_EEMB), lambda i: (i, 0)),
            pl.BlockSpec((BLK_E, D_ATTR), lambda i: (i, 0)),
            pl.BlockSpec((2 * D_FEAT, D_ATTR * D_FEAT), lambda i: (0, 0)),
            pl.BlockSpec((D_EEMB, D_ATTR * D_FEAT), lambda i: (0, 0)),
            pl.BlockSpec((D_FEAT, D_ATTR * D_FEAT), lambda i: (0, 0)),
        ],
        out_specs=pl.BlockSpec((BLK_E, D_FEAT), lambda i: (i, 0)),
        out_shape=jax.ShapeDtypeStruct((E_PAD, D_FEAT), jnp.float32),
    )(xi, xj, emb, att, wab, w1c, w2c)


BLK_N = 2000


def _upd_body(f_ref, pa_ref, pb_ref, na_ref, w3a, w3b, w4, out_ref):
    f = f_ref[...]
    msg = pa_ref[...] + pb_ref[...]
    na = na_ref[...]
    acc = jnp.zeros((BLK_N, D_FEAT), jnp.float32)
    for g in range(D_ATTR):
        t = jnp.dot(f, w3a[g], preferred_element_type=jnp.float32) + jnp.dot(
            msg, w3b[g], preferred_element_type=jnp.float32
        )
        acc = acc + na[:, g : g + 1] * t
    u = _silu(acc * INV3)
    acc2 = jnp.zeros((BLK_N, D_FEAT), jnp.float32)
    for g in range(D_ATTR):
        acc2 = acc2 + na[:, g : g + 1] * jnp.dot(
            u, w4[g], preferred_element_type=jnp.float32
        )
    out_ref[...] = acc2 * INV4 + f


def _update_tc(feats, pa, pb, nattr, w3a, w3b, w4):
    n_blk = N_NODES // BLK_N
    return pl.pallas_call(
        _upd_body,
        grid=(n_blk,),
        in_specs=[
            pl.BlockSpec((BLK_N, D_FEAT), lambda i: (i, 0)),
            pl.BlockSpec((BLK_N, D_FEAT), lambda i: (i, 0)),
            pl.BlockSpec((BLK_N, D_FEAT), lambda i: (i, 0)),
            pl.BlockSpec((BLK_N, D_ATTR), lambda i: (i, 0)),
            pl.BlockSpec((D_ATTR, D_FEAT, D_FEAT), lambda i: (0, 0, 0)),
            pl.BlockSpec((D_ATTR, D_FEAT, D_FEAT), lambda i: (0, 0, 0)),
            pl.BlockSpec((D_ATTR, D_FEAT, D_FEAT), lambda i: (0, 0, 0)),
        ],
        out_specs=pl.BlockSpec((BLK_N, D_FEAT), lambda i: (i, 0)),
        out_shape=jax.ShapeDtypeStruct((N_NODES, D_FEAT), jnp.float32),
    )(feats, pa, pb, nattr, w3a, w3b, w4)


def kernel(node_feats, node_attrs, edge_embedding, edge_attrs, edge_index, batch, W1, W2, W3, W4):
    del batch
    pad = E_PAD - N_EDGES
    src = edge_index[0].astype(jnp.int32)
    dst = edge_index[1].astype(jnp.int32)
    trash = N_PAD - 1
    dst2 = jnp.pad(dst, (0, pad), constant_values=trash).reshape(NCH_PAD, CH)
    src2 = jnp.pad(src, (0, pad), constant_values=trash).reshape(NCH_PAD, CH)

    feats_bf = jnp.pad(
        node_feats.astype(jnp.bfloat16), ((0, N_PAD - N_NODES), (0, 0))
    )
    feats_pk = lax.bitcast_convert_type(feats_bf.reshape(N_PAD, DP, 2), jnp.int32)
    emb_bf = edge_embedding.astype(jnp.bfloat16)
    w1a = jnp.transpose(W1[:D_FEAT], (1, 0, 2)).astype(jnp.bfloat16)
    w1a = jnp.concatenate([w1a[:, 0::2, :], w1a[:, 1::2, :]], axis=1)
    w1b = jnp.transpose(W1[D_FEAT : 2 * D_FEAT], (1, 0, 2)).astype(jnp.bfloat16)
    w1b = jnp.concatenate([w1b[:, 0::2, :], w1b[:, 1::2, :]], axis=1)
    wab = jnp.concatenate(
        [
            jnp.transpose(w1a, (1, 0, 2)).reshape(D_FEAT, D_ATTR * D_FEAT),
            jnp.transpose(w1b, (1, 0, 2)).reshape(D_FEAT, D_ATTR * D_FEAT),
        ],
        axis=0,
    )
    w1c = jnp.transpose(W1[2 * D_FEAT :], (1, 0, 2)).astype(jnp.bfloat16)
    w1c = jnp.transpose(w1c, (1, 0, 2)).reshape(D_EEMB, D_ATTR * D_FEAT)
    w2 = jnp.transpose(W2, (1, 0, 2)).astype(jnp.bfloat16)
    w2c = jnp.transpose(w2, (1, 0, 2)).reshape(D_FEAT, D_ATTR * D_FEAT)
    w3a = jnp.transpose(W3[:D_FEAT], (1, 0, 2))
    w3b = jnp.transpose(W3[D_FEAT:], (1, 0, 2))
    w4 = jnp.transpose(W4, (1, 0, 2))

    gather_sc, scatter_sc = _sc_kernels()
    xi, xj = gather_sc(feats_pk, dst2, src2)
    msgs = _messages_tc(xi, xj, emb_bf, edge_attrs, wab, w1c, w2c)
    partials = scatter_sc(msgs, dst2, jnp.zeros((N_PAD, D_FEAT), jnp.float32))
    return _update_tc(
        node_feats, partials[0, :N_NODES], partials[1, :N_NODES], node_attrs, w3a, w3b, w4
    )


# speedup vs baseline: 2.7591x; 1.0581x over previous
import functools

import jax
import jax.numpy as jnp
from jax import lax
from jax.experimental import pallas as pl
from jax.experimental.pallas import tpu as pltpu
from jax.experimental.pallas import tpu_sc as plsc

N_NODES = 10000
N_EDGES = 160000
D_FEAT = 128
D_EEMB = 16
D_ATTR = 4

NC = 2
NS = 16
NW = NC * NS

CH = 128
E_PAD = 163840
NCH_PAD = E_PAD // CH
NCHUNK = 40
E_HALF = E_PAD // 2
NCH_HALF = NCH_PAD // 2
NCHUNK_G = 20
N_PAD = 10240
RPT = N_PAD // NS
DP = D_FEAT // 2

INV1 = 1.0 / float((272 * 4) ** 0.5)
INV2 = 1.0 / float((128 * 4) ** 0.5)
INV3 = 1.0 / float((256 * 4) ** 0.5)
INV4 = 1.0 / float((512) ** 0.5)


NBUF = 4


def _gather_sc_body(feats, dst2, src2, xi_out, xj_out, idx_d, idx_s, rows, sem_g, sem_w):
    wid = lax.axis_index("s") * NC + lax.axis_index("c")
    cs = wid * NCHUNK_G
    b0 = (cs // 8) * 8
    roff = cs - b0
    pltpu.sync_copy(dst2.at[pl.ds(b0, 24)], idx_d)
    pltpu.sync_copy(src2.at[pl.ds(b0, 24)], idx_s)

    def _fire(j, b):
        idx = idx_d if b % 2 == 0 else idx_s
        return pltpu.async_copy(feats.at[idx.at[roff + j]], rows.at[b], sem_g)

    def _write(j, b):
        out = xi_out if b % 2 == 0 else xj_out
        return pltpu.async_copy(
            rows.at[b], out.at[pl.ds((cs + j) * CH, CH)], sem_w
        )

    def _drain_write(b):
        pltpu.make_async_copy(xi_out.at[pl.ds(0, CH)], rows.at[b], sem_w).wait()

    @pl.loop(0, NCHUNK_G, step=2)
    def _sup(j0):
        gathers = [_fire(j0 + b // 2, b) for b in range(NBUF)]

        @pl.when(j0 > 0)
        def _():
            for b in range(NBUF):
                _drain_write(b)

        for b in range(NBUF):
            gathers[b].wait()
            _write(j0 + b // 2, b)

    for b in range(NBUF):
        _drain_write(b)


def _scatter_sc_body(msgs0, msgs1, dst2, zeros, out, shared, idx_v, rows_v):
    c = lax.axis_index("c")
    s = lax.axis_index("s")
    pltpu.sync_copy(zeros.at[pl.ds(s * RPT, RPT)], shared.at[pl.ds(s * RPT, RPT)])
    cs = s * NCHUNK
    pltpu.sync_copy(dst2.at[pl.ds(c * NCH_HALF + cs, NCHUNK)], idx_v)
    plsc.subcore_barrier()

    @pl.loop(0, NCHUNK)
    def _chunk(j):
        @pl.when(c == 0)
        def _():
            pltpu.sync_copy(msgs0.at[pl.ds((cs + j) * CH, CH)], rows_v)

        @pl.when(c == 1)
        def _():
            pltpu.sync_copy(msgs1.at[pl.ds((cs + j) * CH, CH)], rows_v)

        pltpu.sync_copy(rows_v, shared.at[idx_v.at[j]], add=True)

    plsc.subcore_barrier()
    pltpu.sync_copy(shared.at[pl.ds(s * RPT, RPT)], out.at[c, pl.ds(s * RPT, RPT)])


@functools.lru_cache(maxsize=None)
def _sc_kernels():
    mesh = plsc.VectorSubcoreMesh(
        core_axis_name="c", subcore_axis_name="s", num_cores=NC, num_subcores=NS
    )
    gather = pl.kernel(
        _gather_sc_body,
        compiler_params=pltpu.CompilerParams(use_tc_tiling_on_sc=False),
        out_type=(
            jax.ShapeDtypeStruct((E_HALF, DP), jnp.int32),
            jax.ShapeDtypeStruct((E_HALF, DP), jnp.int32),
        ),
        mesh=mesh,
        scratch_types=[
            pltpu.VMEM((24, CH), jnp.int32),
            pltpu.VMEM((24, CH), jnp.int32),
            pltpu.VMEM((NBUF, CH, DP), jnp.int32),
            pltpu.SemaphoreType.DMA,
            pltpu.SemaphoreType.DMA,
        ],
    )
    scatter = pl.kernel(
        _scatter_sc_body,
        out_type=jax.ShapeDtypeStruct((NC, N_PAD, D_FEAT), jnp.float32),
        mesh=mesh,
        scratch_types=[
            pltpu.VMEM_SHARED((N_PAD, D_FEAT), jnp.float32),
            pltpu.VMEM((NCHUNK, CH), jnp.int32),
            pltpu.VMEM((CH, D_FEAT), jnp.float32),
        ],
    )
    return gather, scatter


BLK_E = 1024


def _silu(x):
    return x * jax.nn.sigmoid(x)


def _unpack_bf16(x):
    lo = lax.bitcast_convert_type(x << 16, jnp.float32)
    hi = lax.bitcast_convert_type(x & jnp.int32(-65536), jnp.float32)
    return jnp.concatenate(
        [lo.astype(jnp.bfloat16), hi.astype(jnp.bfloat16)], axis=1
    )


def _msg_body(xi_ref, xj_ref, emb_ref, att_ref, wab, w1c, w2c, out_ref):
    xi = _unpack_bf16(xi_ref[...])
    xj = _unpack_bf16(xj_ref[...])
    emb = emb_ref[...]
    a = att_ref[...]
    xcat = jnp.concatenate([xi, xj], axis=1)
    t = jnp.dot(xcat, wab[...], preferred_element_type=jnp.float32) + jnp.dot(
        emb, w1c[...], preferred_element_type=jnp.float32
    )
    acc = jnp.zeros((BLK_E, D_FEAT), jnp.float32)
    for g in range(D_ATTR):
        acc = acc + a[:, g : g + 1] * t[:, g * D_FEAT : (g + 1) * D_FEAT]
    m1 = _silu(acc * INV1).astype(jnp.bfloat16)
    t2 = jnp.dot(m1, w2c[...], preferred_element_type=jnp.float32)
    acc2 = jnp.zeros((BLK_E, D_FEAT), jnp.float32)
    for g in range(D_ATTR):
        acc2 = acc2 + a[:, g : g + 1] * t2[:, g * D_FEAT : (g + 1) * D_FEAT]
    out_ref[...] = _silu(acc2 * INV2)


def _messages_tc(n_real, xi, xj, emb, att, wab, w1c, w2c):
    n_blk = (n_real + BLK_E - 1) // BLK_E
    return pl.pallas_call(
        _msg_body,
        grid=(n_blk,),
        in_specs=[
            pl.BlockSpec((BLK_E, DP), lambda i: (i, 0)),
            pl.BlockSpec((BLK_E, DP), lambda i: (i, 0)),
            pl.BlockSpec((BLK_E, D_EEMB), lambda i: (i, 0)),
            pl.BlockSpec((BLK_E, D_ATTR), lambda i: (i, 0)),
            pl.BlockSpec((2 * D_FEAT, D_ATTR * D_FEAT), lambda i: (0, 0)),
            pl.BlockSpec((D_EEMB, D_ATTR * D_FEAT), lambda i: (0, 0)),
            pl.BlockSpec((D_FEAT, D_ATTR * D_FEAT), lambda i: (0, 0)),
        ],
        out_specs=pl.BlockSpec((BLK_E, D_FEAT), lambda i: (i, 0)),
        out_shape=jax.ShapeDtypeStruct((E_HALF, D_FEAT), jnp.float32),
    )(xi, xj, emb, att, wab, w1c, w2c)


BLK_N = 2000


def _upd_body(f_ref, pa_ref, pb_ref, na_ref, w3a, w3b, w4, out_ref):
    f = f_ref[...]
    msg = pa_ref[...] + pb_ref[...]
    na = na_ref[...]
    acc = jnp.zeros((BLK_N, D_FEAT), jnp.float32)
    for g in range(D_ATTR):
        t = jnp.dot(f, w3a[g], preferred_element_type=jnp.float32) + jnp.dot(
            msg, w3b[g], preferred_element_type=jnp.float32
        )
        acc = acc + na[:, g : g + 1] * t
    u = _silu(acc * INV3)
    acc2 = jnp.zeros((BLK_N, D_FEAT), jnp.float32)
    for g in range(D_ATTR):
        acc2 = acc2 + na[:, g : g + 1] * jnp.dot(
            u, w4[g], preferred_element_type=jnp.float32
        )
    out_ref[...] = acc2 * INV4 + f


def _update_tc(feats, pa, pb, nattr, w3a, w3b, w4):
    n_blk = N_NODES // BLK_N
    return pl.pallas_call(
        _upd_body,
        grid=(n_blk,),
        in_specs=[
            pl.BlockSpec((BLK_N, D_FEAT), lambda i: (i, 0)),
            pl.BlockSpec((BLK_N, D_FEAT), lambda i: (i, 0)),
            pl.BlockSpec((BLK_N, D_FEAT), lambda i: (i, 0)),
            pl.BlockSpec((BLK_N, D_ATTR), lambda i: (i, 0)),
            pl.BlockSpec((D_ATTR, D_FEAT, D_FEAT), lambda i: (0, 0, 0)),
            pl.BlockSpec((D_ATTR, D_FEAT, D_FEAT), lambda i: (0, 0, 0)),
            pl.BlockSpec((D_ATTR, D_FEAT, D_FEAT), lambda i: (0, 0, 0)),
        ],
        out_specs=pl.BlockSpec((BLK_N, D_FEAT), lambda i: (i, 0)),
        out_shape=jax.ShapeDtypeStruct((N_NODES, D_FEAT), jnp.float32),
    )(feats, pa, pb, nattr, w3a, w3b, w4)


def kernel(node_feats, node_attrs, edge_embedding, edge_attrs, edge_index, batch, W1, W2, W3, W4):
    del batch
    pad = E_PAD - N_EDGES
    src = edge_index[0].astype(jnp.int32)
    dst = edge_index[1].astype(jnp.int32)
    trash = N_PAD - 1
    dst2 = jnp.pad(dst, (0, pad), constant_values=trash).reshape(NCH_PAD, CH)
    src2 = jnp.pad(src, (0, pad), constant_values=trash).reshape(NCH_PAD, CH)

    feats_bf = jnp.pad(
        node_feats.astype(jnp.bfloat16), ((0, N_PAD - N_NODES), (0, 0))
    )
    feats_pk = lax.bitcast_convert_type(feats_bf.reshape(N_PAD, DP, 2), jnp.int32)
    emb_bf = edge_embedding.astype(jnp.bfloat16)
    w1a = jnp.transpose(W1[:D_FEAT], (1, 0, 2)).astype(jnp.bfloat16)
    w1a = jnp.concatenate([w1a[:, 0::2, :], w1a[:, 1::2, :]], axis=1)
    w1b = jnp.transpose(W1[D_FEAT : 2 * D_FEAT], (1, 0, 2)).astype(jnp.bfloat16)
    w1b = jnp.concatenate([w1b[:, 0::2, :], w1b[:, 1::2, :]], axis=1)
    wab = jnp.concatenate(
        [
            jnp.transpose(w1a, (1, 0, 2)).reshape(D_FEAT, D_ATTR * D_FEAT),
            jnp.transpose(w1b, (1, 0, 2)).reshape(D_FEAT, D_ATTR * D_FEAT),
        ],
        axis=0,
    )
    w1c = jnp.transpose(W1[2 * D_FEAT :], (1, 0, 2)).astype(jnp.bfloat16)
    w1c = jnp.transpose(w1c, (1, 0, 2)).reshape(D_EEMB, D_ATTR * D_FEAT)
    w2 = jnp.transpose(W2, (1, 0, 2)).astype(jnp.bfloat16)
    w2c = jnp.transpose(w2, (1, 0, 2)).reshape(D_FEAT, D_ATTR * D_FEAT)
    w3a = jnp.transpose(W3[:D_FEAT], (1, 0, 2))
    w3b = jnp.transpose(W3[D_FEAT:], (1, 0, 2))
    w4 = jnp.transpose(W4, (1, 0, 2))

    gather_sc, scatter_sc = _sc_kernels()
    xi0, xj0 = gather_sc(feats_pk, dst2[:NCH_HALF], src2[:NCH_HALF])
    xi1, xj1 = gather_sc(feats_pk, dst2[NCH_HALF:], src2[NCH_HALF:])
    msgs0 = _messages_tc(
        E_HALF, xi0, xj0, emb_bf[:E_HALF], edge_attrs[:E_HALF], wab, w1c, w2c
    )
    msgs1 = _messages_tc(
        N_EDGES - E_HALF, xi1, xj1, emb_bf[E_HALF:], edge_attrs[E_HALF:],
        wab, w1c, w2c,
    )
    partials = scatter_sc(
        msgs0, msgs1, dst2, jnp.zeros((N_PAD, D_FEAT), jnp.float32)
    )
    return _update_tc(
        node_feats, partials[0, :N_NODES], partials[1, :N_NODES], node_attrs, w3a, w3b, w4
    )


# speedup vs baseline: 2.8019x; 1.0155x over previous
import functools

import jax
import jax.numpy as jnp
from jax import lax
from jax.experimental import pallas as pl
from jax.experimental.pallas import tpu as pltpu
from jax.experimental.pallas import tpu_sc as plsc

N_NODES = 10000
N_EDGES = 160000
D_FEAT = 128
D_EEMB = 16
D_ATTR = 4

NC = 2
NS = 16
NW = NC * NS

CH = 128
E_PAD = 163840
NCH_PAD = E_PAD // CH
NCHUNK = 40
E_HALF = E_PAD // 2
NCH_HALF = NCH_PAD // 2
NCHUNK_G = 20
N_PAD = 10240
RPT = N_PAD // NS
DP = D_FEAT // 2

INV1 = 1.0 / float((272 * 4) ** 0.5)
INV2 = 1.0 / float((128 * 4) ** 0.5)
INV3 = 1.0 / float((256 * 4) ** 0.5)
INV4 = 1.0 / float((512) ** 0.5)


NBUF = 4


def _gather_sc_body(feats, dst2, src2, xi_out, xj_out, idx_d, idx_s, rows, sem_g, sem_w):
    c = lax.axis_index("c")
    s = lax.axis_index("s")
    cs = s * 2 * NCHUNK_G + c * 24
    n = 24 - 8 * c
    b0 = jnp.minimum(cs, NCH_HALF - 24)
    roff = cs - b0
    pltpu.sync_copy(dst2.at[pl.ds(b0, 24)], idx_d)
    pltpu.sync_copy(src2.at[pl.ds(b0, 24)], idx_s)

    def _fire(j, b):
        idx = idx_d if b % 2 == 0 else idx_s
        return pltpu.async_copy(feats.at[idx.at[roff + j]], rows.at[b], sem_g)

    def _write(j, b):
        out = xi_out if b % 2 == 0 else xj_out
        return pltpu.async_copy(
            rows.at[b], out.at[pl.ds((cs + j) * CH, CH)], sem_w
        )

    def _drain_write(b):
        pltpu.make_async_copy(xi_out.at[pl.ds(0, CH)], rows.at[b], sem_w).wait()

    @pl.loop(0, n, step=2)
    def _sup(j0):
        gathers = [_fire(j0 + b // 2, b) for b in range(NBUF)]

        @pl.when(j0 > 0)
        def _():
            for b in range(NBUF):
                _drain_write(b)

        for b in range(NBUF):
            gathers[b].wait()
            _write(j0 + b // 2, b)

    for b in range(NBUF):
        _drain_write(b)


def _scatter_sc_body(msgs0, msgs1, dst2, zeros, out, shared, idx_v, rows_v):
    c = lax.axis_index("c")
    s = lax.axis_index("s")
    pltpu.sync_copy(zeros.at[pl.ds(s * RPT, RPT)], shared.at[pl.ds(s * RPT, RPT)])
    cs = s * NCHUNK
    pltpu.sync_copy(dst2.at[pl.ds(c * NCH_HALF + cs, NCHUNK)], idx_v)
    plsc.subcore_barrier()

    @pl.loop(0, NCHUNK)
    def _chunk(j):
        @pl.when(c == 0)
        def _():
            pltpu.sync_copy(msgs0.at[pl.ds((cs + j) * CH, CH)], rows_v)

        @pl.when(c == 1)
        def _():
            pltpu.sync_copy(msgs1.at[pl.ds((cs + j) * CH, CH)], rows_v)

        pltpu.sync_copy(rows_v, shared.at[idx_v.at[j]], add=True)

    plsc.subcore_barrier()
    pltpu.sync_copy(shared.at[pl.ds(s * RPT, RPT)], out.at[c, pl.ds(s * RPT, RPT)])


@functools.lru_cache(maxsize=None)
def _sc_kernels():
    mesh = plsc.VectorSubcoreMesh(
        core_axis_name="c", subcore_axis_name="s", num_cores=NC, num_subcores=NS
    )
    gather = pl.kernel(
        _gather_sc_body,
        compiler_params=pltpu.CompilerParams(use_tc_tiling_on_sc=False),
        out_type=(
            jax.ShapeDtypeStruct((E_HALF, DP), jnp.int32),
            jax.ShapeDtypeStruct((E_HALF, DP), jnp.int32),
        ),
        mesh=mesh,
        scratch_types=[
            pltpu.VMEM((24, CH), jnp.int32),
            pltpu.VMEM((24, CH), jnp.int32),
            pltpu.VMEM((NBUF, CH, DP), jnp.int32),
            pltpu.SemaphoreType.DMA,
            pltpu.SemaphoreType.DMA,
        ],
    )
    scatter = pl.kernel(
        _scatter_sc_body,
        out_type=jax.ShapeDtypeStruct((NC, N_PAD, D_FEAT), jnp.float32),
        mesh=mesh,
        scratch_types=[
            pltpu.VMEM_SHARED((N_PAD, D_FEAT), jnp.float32),
            pltpu.VMEM((NCHUNK, CH), jnp.int32),
            pltpu.VMEM((CH, D_FEAT), jnp.float32),
        ],
    )
    return gather, scatter


BLK_E = 1024


def _silu(x):
    return x * jax.nn.sigmoid(x)


def _unpack_bf16(x):
    lo = lax.bitcast_convert_type(x << 16, jnp.float32)
    hi = lax.bitcast_convert_type(x & jnp.int32(-65536), jnp.float32)
    return jnp.concatenate(
        [lo.astype(jnp.bfloat16), hi.astype(jnp.bfloat16)], axis=1
    )


def _msg_body(xi_ref, xj_ref, emb_ref, att_ref, wab, w1c, w2c, out_ref):
    xi = _unpack_bf16(xi_ref[...])
    xj = _unpack_bf16(xj_ref[...])
    emb = emb_ref[...]
    a = att_ref[...]
    xcat = jnp.concatenate([xi, xj], axis=1)
    t = jnp.dot(xcat, wab[...], preferred_element_type=jnp.float32) + jnp.dot(
        emb, w1c[...], preferred_element_type=jnp.float32
    )
    acc = jnp.zeros((BLK_E, D_FEAT), jnp.float32)
    for g in range(D_ATTR):
        acc = acc + a[:, g : g + 1] * t[:, g * D_FEAT : (g + 1) * D_FEAT]
    m1 = _silu(acc * INV1).astype(jnp.bfloat16)
    t2 = jnp.dot(m1, w2c[...], preferred_element_type=jnp.float32)
    acc2 = jnp.zeros((BLK_E, D_FEAT), jnp.float32)
    for g in range(D_ATTR):
        acc2 = acc2 + a[:, g : g + 1] * t2[:, g * D_FEAT : (g + 1) * D_FEAT]
    out_ref[...] = _silu(acc2 * INV2)


def _messages_tc(n_real, xi, xj, emb, att, wab, w1c, w2c):
    n_blk = (n_real + BLK_E - 1) // BLK_E
    return pl.pallas_call(
        _msg_body,
        grid=(n_blk,),
        in_specs=[
            pl.BlockSpec((BLK_E, DP), lambda i: (i, 0)),
            pl.BlockSpec((BLK_E, DP), lambda i: (i, 0)),
            pl.BlockSpec((BLK_E, D_EEMB), lambda i: (i, 0)),
            pl.BlockSpec((BLK_E, D_ATTR), lambda i: (i, 0)),
            pl.BlockSpec((2 * D_FEAT, D_ATTR * D_FEAT), lambda i: (0, 0)),
            pl.BlockSpec((D_EEMB, D_ATTR * D_FEAT), lambda i: (0, 0)),
            pl.BlockSpec((D_FEAT, D_ATTR * D_FEAT), lambda i: (0, 0)),
        ],
        out_specs=pl.BlockSpec((BLK_E, D_FEAT), lambda i: (i, 0)),
        out_shape=jax.ShapeDtypeStruct((E_HALF, D_FEAT), jnp.float32),
    )(xi, xj, emb, att, wab, w1c, w2c)


BLK_N = 2000


def _upd_body(f_ref, pa_ref, pb_ref, na_ref, w3a, w3b, w4, out_ref):
    f = f_ref[...]
    msg = pa_ref[...] + pb_ref[...]
    na = na_ref[...]
    acc = jnp.zeros((BLK_N, D_FEAT), jnp.float32)
    for g in range(D_ATTR):
        t = jnp.dot(f, w3a[g], preferred_element_type=jnp.float32) + jnp.dot(
            msg, w3b[g], preferred_element_type=jnp.float32
        )
        acc = acc + na[:, g : g + 1] * t
    u = _silu(acc * INV3)
    acc2 = jnp.zeros((BLK_N, D_FEAT), jnp.float32)
    for g in range(D_ATTR):
        acc2 = acc2 + na[:, g : g + 1] * jnp.dot(
            u, w4[g], preferred_element_type=jnp.float32
        )
    out_ref[...] = acc2 * INV4 + f


def _update_tc(feats, pa, pb, nattr, w3a, w3b, w4):
    n_blk = N_NODES // BLK_N
    return pl.pallas_call(
        _upd_body,
        grid=(n_blk,),
        in_specs=[
            pl.BlockSpec((BLK_N, D_FEAT), lambda i: (i, 0)),
            pl.BlockSpec((BLK_N, D_FEAT), lambda i: (i, 0)),
            pl.BlockSpec((BLK_N, D_FEAT), lambda i: (i, 0)),
            pl.BlockSpec((BLK_N, D_ATTR), lambda i: (i, 0)),
            pl.BlockSpec((D_ATTR, D_FEAT, D_FEAT), lambda i: (0, 0, 0)),
            pl.BlockSpec((D_ATTR, D_FEAT, D_FEAT), lambda i: (0, 0, 0)),
            pl.BlockSpec((D_ATTR, D_FEAT, D_FEAT), lambda i: (0, 0, 0)),
        ],
        out_specs=pl.BlockSpec((BLK_N, D_FEAT), lambda i: (i, 0)),
        out_shape=jax.ShapeDtypeStruct((N_NODES, D_FEAT), jnp.float32),
    )(feats, pa, pb, nattr, w3a, w3b, w4)


def kernel(node_feats, node_attrs, edge_embedding, edge_attrs, edge_index, batch, W1, W2, W3, W4):
    del batch
    pad = E_PAD - N_EDGES
    src = edge_index[0].astype(jnp.int32)
    dst = edge_index[1].astype(jnp.int32)
    trash = N_PAD - 1
    dst2 = jnp.pad(dst, (0, pad), constant_values=trash).reshape(NCH_PAD, CH)
    src2 = jnp.pad(src, (0, pad), constant_values=trash).reshape(NCH_PAD, CH)

    feats_bf = jnp.pad(
        node_feats.astype(jnp.bfloat16), ((0, N_PAD - N_NODES), (0, 0))
    )
    feats_pk = lax.bitcast_convert_type(feats_bf.reshape(N_PAD, DP, 2), jnp.int32)
    emb_bf = edge_embedding.astype(jnp.bfloat16)
    w1a = jnp.transpose(W1[:D_FEAT], (1, 0, 2)).astype(jnp.bfloat16)
    w1a = jnp.concatenate([w1a[:, 0::2, :], w1a[:, 1::2, :]], axis=1)
    w1b = jnp.transpose(W1[D_FEAT : 2 * D_FEAT], (1, 0, 2)).astype(jnp.bfloat16)
    w1b = jnp.concatenate([w1b[:, 0::2, :], w1b[:, 1::2, :]], axis=1)
    wab = jnp.concatenate(
        [
            jnp.transpose(w1a, (1, 0, 2)).reshape(D_FEAT, D_ATTR * D_FEAT),
            jnp.transpose(w1b, (1, 0, 2)).reshape(D_FEAT, D_ATTR * D_FEAT),
        ],
        axis=0,
    )
    w1c = jnp.transpose(W1[2 * D_FEAT :], (1, 0, 2)).astype(jnp.bfloat16)
    w1c = jnp.transpose(w1c, (1, 0, 2)).reshape(D_EEMB, D_ATTR * D_FEAT)
    w2 = jnp.transpose(W2, (1, 0, 2)).astype(jnp.bfloat16)
    w2c = jnp.transpose(w2, (1, 0, 2)).reshape(D_FEAT, D_ATTR * D_FEAT)
    w3a = jnp.transpose(W3[:D_FEAT], (1, 0, 2))
    w3b = jnp.transpose(W3[D_FEAT:], (1, 0, 2))
    w4 = jnp.transpose(W4, (1, 0, 2))

    gather_sc, scatter_sc = _sc_kernels()
    xi0, xj0 = gather_sc(feats_pk, dst2[:NCH_HALF], src2[:NCH_HALF])
    xi1, xj1 = gather_sc(feats_pk, dst2[NCH_HALF:], src2[NCH_HALF:])
    msgs0 = _messages_tc(
        E_HALF, xi0, xj0, emb_bf[:E_HALF], edge_attrs[:E_HALF], wab, w1c, w2c
    )
    msgs1 = _messages_tc(
        N_EDGES - E_HALF, xi1, xj1, emb_bf[E_HALF:], edge_attrs[E_HALF:],
        wab, w1c, w2c,
    )
    partials = scatter_sc(
        msgs0, msgs1, dst2, jnp.zeros((N_PAD, D_FEAT), jnp.float32)
    )
    return _update_tc(
        node_feats, partials[0, :N_NODES], partials[1, :N_NODES], node_attrs, w3a, w3b, w4
    )
